# baseline (speedup 1.0000x reference)
import functools
import jax
import jax.numpy as jnp
from jax import lax
from jax.experimental import pallas as pl
from jax.experimental.pallas import tpu as pltpu
from jax.experimental.pallas import tpu_sc as plsc

_B, _S, _D = 8, 2048, 768
_NS = 64
_DQ = _D // 8
_CHUNK = 512
_NCH = _S // _CHUNK
_NGRP = _CHUNK // 16


def _proj_body(x_ref, wq_ref, bq_ref, wk_ref, bk_ref, qt_ref, k_ref):
    xb = x_ref[0]
    qt = lax.dot_general(wq_ref[:], xb, (((0,), (1,)), ((), ())),
                         preferred_element_type=jnp.float32)
    qt_ref[0] = qt + bq_ref[:]
    k_ref[0] = jnp.dot(xb, wk_ref[:], preferred_element_type=jnp.float32) + bk_ref[:]


def _projections(x, Wq, bq, Wk, bk):
    nb = x.shape[0]
    return pl.pallas_call(
        _proj_body,
        grid=(nb,),
        in_specs=[
            pl.BlockSpec((1, _S, _D), lambda b: (b, 0, 0)),
            pl.BlockSpec((_D, _DQ), lambda b: (0, 0)),
            pl.BlockSpec((_DQ, 1), lambda b: (0, 0)),
            pl.BlockSpec((_D, _DQ), lambda b: (0, 0)),
            pl.BlockSpec((1, _DQ), lambda b: (0, 0)),
        ],
        out_specs=[
            pl.BlockSpec((1, _DQ, _S), lambda b: (b, 0, 0)),
            pl.BlockSpec((1, _S, _DQ), lambda b: (b, 0, 0)),
        ],
        out_shape=[
            jax.ShapeDtypeStruct((nb, _DQ, _S), jnp.float32),
            jax.ShapeDtypeStruct((nb, _S, _DQ), jnp.float32),
        ],
        compiler_params=pltpu.CompilerParams(
            dimension_semantics=("arbitrary",),
        ),
    )(x, Wq, bq.reshape(_DQ, 1), Wk, bk.reshape(1, _DQ))


def _iota16():
    return lax.iota(jnp.int32, 16)


def _splat_f(v):
    return jnp.zeros((16,), jnp.float32) + v


def _splat_i(v):
    return jnp.zeros((16,), jnp.int32) + v


def _make_sc_body(nb):
    spc = nb // 2
    nch = 16 // spc
    ctok = _S // nch
    ngrp = ctok // 16
    nrows = ctok // 128

    def _sc_body(q_hbm, k_hbm, seg_hbm, a_hbm,
                 segf_v, seg2_v, kq_v, qt_v, ksum_v, s_v, e_v, red_v, part_v,
                 a_v, ksum_sh, stage_sh, sem0, sem1, sem2):
        cid = lax.axis_index("c")
        sid = lax.axis_index("s")
        slab = sid // nch
        chunk = sid % nch
        b = cid * spc + slab
        tok0 = chunk * ctok

        seg_cp = pltpu.async_copy(seg_hbm.at[b, pl.ds(tok0, ctok)], segf_v, sem0)
        k_cp = pltpu.async_copy(k_hbm.at[b, pl.ds(tok0, ctok), :], kq_v, sem1)
        q_cp = pltpu.async_copy(q_hbm.at[b, :, pl.ds(tok0, ctok)], qt_v, sem2)

        seg_cp.wait()
        for r in range(nrows):
            for cc in range(8):
                v = segf_v[pl.ds(r * 128 + cc * 16, 16)] + slab * _NS
                seg2_v[r, cc * 16:(cc + 1) * 16] = v

        @pl.when(chunk == 0)
        def _():
            for rr in range(_NS):
                for cc in range(_DQ // 16):
                    ksum_v[rr, cc * 16:(cc + 1) * 16] = jnp.zeros((16,), jnp.float32)
            pltpu.sync_copy(ksum_v, ksum_sh.at[pl.ds(slab * _NS, _NS), :])
        plsc.subcore_barrier()

        k_cp.wait()
        adds = [pltpu.async_copy(kq_v.at[pl.ds(r * 128, 128), :],
                                 ksum_sh.at[seg2_v.at[r]], sem1, add=True)
                for r in range(nrows)]
        for cp in adds:
            cp.wait()
        plsc.subcore_barrier()

        pltpu.sync_copy(ksum_sh.at[pl.ds(slab * _NS, _NS), :], ksum_v)

        q_cp.wait()

        def _s_group(g, carry):
            segg = segf_v[pl.ds(g * 16, 16)]
            acc = jnp.zeros((16,), jnp.float32)
            for j in range(_DQ):
                qv = qt_v[j, pl.ds(g * 16, 16)]
                kg = plsc.load_gather(ksum_v, [segg, _splat_i(j)])
                acc = acc + qv * kg
            s_v[pl.ds(g * 16, 16)] = acc
            return carry

        lax.fori_loop(0, ngrp, _s_group, 0)

        lo = jnp.min(segf_v[pl.ds(0, 16)])
        hi = jnp.max(segf_v[pl.ds((ngrp - 1) * 16, 16)])

        lane0 = _iota16() == 0
        neg_inf = jnp.float32(-jnp.inf)

        def _reduce_chunks(op):
            pltpu.sync_copy(red_v, stage_sh.at[slab, chunk])
            plsc.subcore_barrier()
            pltpu.sync_copy(stage_sh.at[slab], part_v)
            plsc.subcore_barrier()
            for cc in range(_NS // 16):
                v = part_v[0, cc * 16:(cc + 1) * 16]
                for r in range(1, nch):
                    v = op(v, part_v[r, cc * 16:(cc + 1) * 16])
                red_v[pl.ds(cc * 16, 16)] = v

        for cc in range(_NS // 16):
            red_v[pl.ds(cc * 16, 16)] = _splat_f(neg_inf)

        def _max_ns(ns, carry):
            mv = _splat_f(neg_inf)
            for g in range(ngrp):
                segg = segf_v[pl.ds(g * 16, 16)]
                sx = s_v[pl.ds(g * 16, 16)]
                mv = jnp.maximum(mv, jnp.where(segg == ns, sx, neg_inf))
            m = jnp.max(mv)
            plsc.store_scatter(red_v, [_splat_i(ns)], _splat_f(m), mask=lane0)
            return carry

        lax.fori_loop(lo, hi + 1, _max_ns, 0)

        _reduce_chunks(jnp.maximum)

        def _e_group(g, carry):
            segg = segf_v[pl.ds(g * 16, 16)]
            sx = s_v[pl.ds(g * 16, 16)]
            mx = plsc.load_gather(red_v, [segg])
            e_v[pl.ds(g * 16, 16)] = jnp.exp(sx - mx)
            return carry

        lax.fori_loop(0, ngrp, _e_group, 0)

        for cc in range(_NS // 16):
            red_v[pl.ds(cc * 16, 16)] = jnp.zeros((16,), jnp.float32)

        def _sum_ns(ns, carry):
            sv = jnp.zeros((16,), jnp.float32)
            for g in range(ngrp):
                segg = segf_v[pl.ds(g * 16, 16)]
                ex = e_v[pl.ds(g * 16, 16)]
                sv = sv + jnp.where(segg == ns, ex, 0.0)
            t = jnp.sum(sv)
            plsc.store_scatter(red_v, [_splat_i(ns)], _splat_f(t), mask=lane0)
            return carry

        lax.fori_loop(lo, hi + 1, _sum_ns, 0)

        _reduce_chunks(jnp.add)

        def _a_group(g, carry):
            segg = segf_v[pl.ds(g * 16, 16)]
            ex = e_v[pl.ds(g * 16, 16)]
            es = plsc.load_gather(red_v, [segg])
            a_v[pl.ds(g * 16, 16)] = ex / es
            return carry

        lax.fori_loop(0, ngrp, _a_group, 0)

        pltpu.sync_copy(a_v, a_hbm.at[b, pl.ds(tok0, ctok)])

    return _sc_body, spc, nch, ctok, nrows


def _sc_weights(q, k, seg):
    nb = q.shape[0]
    body, spc, nch, ctok, nrows = _make_sc_body(nb)
    mesh = plsc.VectorSubcoreMesh(core_axis_name="c", subcore_axis_name="s")
    f = pl.kernel(
        body,
        out_type=jax.ShapeDtypeStruct((nb, _S), jnp.float32),
        mesh=mesh,
        compiler_params=pltpu.CompilerParams(needs_layout_passes=False),
        scratch_types=[
            pltpu.VMEM((ctok,), jnp.int32),
            pltpu.VMEM((nrows, 128), jnp.int32),
            pltpu.VMEM((ctok, _DQ), jnp.float32),
            pltpu.VMEM((_DQ, ctok), jnp.float32),
            pltpu.VMEM((_NS, _DQ), jnp.float32),
            pltpu.VMEM((ctok,), jnp.float32),
            pltpu.VMEM((ctok,), jnp.float32),
            pltpu.VMEM((_NS,), jnp.float32),
            pltpu.VMEM((nch, _NS), jnp.float32),
            pltpu.VMEM((ctok,), jnp.float32),
            pltpu.VMEM_SHARED((spc * _NS, _DQ), jnp.float32),
            pltpu.VMEM_SHARED((spc, nch, _NS), jnp.float32),
            pltpu.SemaphoreType.DMA,
            pltpu.SemaphoreType.DMA,
            pltpu.SemaphoreType.DMA,
        ],
    )
    return f(q, k, seg)


def _pool_body(seg_ref, a_ref, x_ref, out_ref):
    xb = x_ref[0]
    seg_row = seg_ref[0]
    a_row = a_ref[0]
    ids_ns_s = lax.broadcasted_iota(jnp.int32, (_NS, _S), 0)
    onehot = (ids_ns_s == seg_row).astype(jnp.float32)
    out_ref[0] = jnp.dot(onehot * a_row, xb, preferred_element_type=jnp.float32)


def _pool(x, seg3, a):
    nb = x.shape[0]
    return pl.pallas_call(
        _pool_body,
        grid=(nb,),
        in_specs=[
            pl.BlockSpec((1, 1, _S), lambda b: (b, 0, 0)),
            pl.BlockSpec((1, 1, _S), lambda b: (b, 0, 0)),
            pl.BlockSpec((1, _S, _D), lambda b: (b, 0, 0)),
        ],
        out_specs=pl.BlockSpec((1, _NS, _D), lambda b: (b, 0, 0)),
        out_shape=jax.ShapeDtypeStruct((nb, _NS, _D), jnp.float32),
        compiler_params=pltpu.CompilerParams(
            dimension_semantics=("arbitrary",),
        ),
    )(seg3, a.reshape(nb, 1, _S), x)


def kernel(x, sentence_index, Wq, bq, Wk, bk):
    seg = sentence_index.astype(jnp.int32)
    outs = []
    for g in range(2):
        xg = x[g * 4:(g + 1) * 4]
        sg = seg[g * 4:(g + 1) * 4]
        qg, kg = _projections(xg, Wq, bq, Wk, bk)
        ag = _sc_weights(qg, kg, sg)
        outs.append(_pool(xg, sg.reshape(4, 1, _S), ag))
    out = jnp.concatenate(outs, axis=0)
    return out[:, 1:]


# speedup vs baseline: 1.2586x; 1.2586x over previous
import functools
import jax
import jax.numpy as jnp
from jax import lax
from jax.experimental import pallas as pl
from jax.experimental.pallas import tpu as pltpu
from jax.experimental.pallas import tpu_sc as plsc

_B, _S, _D = 8, 2048, 768
_NS = 64
_DQ = _D // 8
_CHUNK = 512
_NCH = _S // _CHUNK
_NGRP = _CHUNK // 16


def _proj_body(x_ref, wq_ref, bq_ref, wk_ref, bk_ref, qt_ref, k_ref):
    xb = x_ref[0]
    qt = lax.dot_general(wq_ref[:], xb, (((0,), (1,)), ((), ())),
                         preferred_element_type=jnp.float32)
    qt_ref[0] = qt + bq_ref[:]
    k_ref[0] = jnp.dot(xb, wk_ref[:], preferred_element_type=jnp.float32) + bk_ref[:]


def _projections(x, Wq, bq, Wk, bk):
    nb = x.shape[0]
    return pl.pallas_call(
        _proj_body,
        grid=(nb,),
        in_specs=[
            pl.BlockSpec((1, _S, _D), lambda b: (b, 0, 0)),
            pl.BlockSpec((_D, _DQ), lambda b: (0, 0)),
            pl.BlockSpec((_DQ, 1), lambda b: (0, 0)),
            pl.BlockSpec((_D, _DQ), lambda b: (0, 0)),
            pl.BlockSpec((1, _DQ), lambda b: (0, 0)),
        ],
        out_specs=[
            pl.BlockSpec((1, _DQ, _S), lambda b: (b, 0, 0)),
            pl.BlockSpec((1, _S, _DQ), lambda b: (b, 0, 0)),
        ],
        out_shape=[
            jax.ShapeDtypeStruct((nb, _DQ, _S), jnp.float32),
            jax.ShapeDtypeStruct((nb, _S, _DQ), jnp.float32),
        ],
        compiler_params=pltpu.CompilerParams(
            dimension_semantics=("arbitrary",),
        ),
    )(x, Wq, bq.reshape(_DQ, 1), Wk, bk.reshape(1, _DQ))


def _iota16():
    return lax.iota(jnp.int32, 16)


def _splat_f(v):
    return jnp.zeros((16,), jnp.float32) + v


def _splat_i(v):
    return jnp.zeros((16,), jnp.int32) + v


def _make_sc_body(nb):
    spc = nb // 2
    nch = 16 // spc
    ctok = _S // nch
    ngrp = ctok // 16
    nrows = ctok // 128

    def _sc_body(q_hbm, k_hbm, seg_hbm, a_hbm,
                 segf_v, seg2_v, kq_v, qt_v, ksum_v, s_v, e_v, red_v, part_v,
                 a_v, ksum_sh, stage_sh, sem0, sem1, sem2):
        cid = lax.axis_index("c")
        sid = lax.axis_index("s")
        slab = sid // nch
        chunk = sid % nch
        b = cid * spc + slab
        tok0 = chunk * ctok

        seg_cp = pltpu.async_copy(seg_hbm.at[b, pl.ds(tok0, ctok)], segf_v, sem0)
        k_cp = pltpu.async_copy(k_hbm.at[b, pl.ds(tok0, ctok), :], kq_v, sem1)
        q_cp = pltpu.async_copy(q_hbm.at[b, :, pl.ds(tok0, ctok)], qt_v, sem2)

        seg_cp.wait()
        for r in range(nrows):
            for cc in range(8):
                v = segf_v[pl.ds(r * 128 + cc * 16, 16)] + slab * _NS
                seg2_v[r, cc * 16:(cc + 1) * 16] = v

        @pl.when(chunk == 0)
        def _():
            for rr in range(_NS):
                for cc in range(_DQ // 16):
                    ksum_v[rr, cc * 16:(cc + 1) * 16] = jnp.zeros((16,), jnp.float32)
            pltpu.sync_copy(ksum_v, ksum_sh.at[pl.ds(slab * _NS, _NS), :])
        plsc.subcore_barrier()

        k_cp.wait()
        adds = [pltpu.async_copy(kq_v.at[pl.ds(r * 128, 128), :],
                                 ksum_sh.at[seg2_v.at[r]], sem1, add=True)
                for r in range(nrows)]
        for cp in adds:
            cp.wait()
        plsc.subcore_barrier()

        pltpu.sync_copy(ksum_sh.at[pl.ds(slab * _NS, _NS), :], ksum_v)

        q_cp.wait()

        def _s_group(g, carry):
            segg = segf_v[pl.ds(g * 16, 16)]
            acc = jnp.zeros((16,), jnp.float32)
            for j in range(_DQ):
                qv = qt_v[j, pl.ds(g * 16, 16)]
                kg = plsc.load_gather(ksum_v, [segg, _splat_i(j)])
                acc = acc + qv * kg
            s_v[pl.ds(g * 16, 16)] = acc
            return carry

        lax.fori_loop(0, ngrp, _s_group, 0)

        lo = jnp.min(segf_v[pl.ds(0, 16)])
        hi = jnp.max(segf_v[pl.ds((ngrp - 1) * 16, 16)])

        lane0 = _iota16() == 0
        neg_inf = jnp.float32(-jnp.inf)

        def _reduce_chunks(op):
            pltpu.sync_copy(red_v, stage_sh.at[slab, chunk])
            plsc.subcore_barrier()
            pltpu.sync_copy(stage_sh.at[slab], part_v)
            plsc.subcore_barrier()
            for cc in range(_NS // 16):
                v = part_v[0, cc * 16:(cc + 1) * 16]
                for r in range(1, nch):
                    v = op(v, part_v[r, cc * 16:(cc + 1) * 16])
                red_v[pl.ds(cc * 16, 16)] = v

        for cc in range(_NS // 16):
            red_v[pl.ds(cc * 16, 16)] = _splat_f(neg_inf)

        def _max_ns(ns, carry):
            mv = _splat_f(neg_inf)
            for g in range(ngrp):
                segg = segf_v[pl.ds(g * 16, 16)]
                sx = s_v[pl.ds(g * 16, 16)]
                mv = jnp.maximum(mv, jnp.where(segg == ns, sx, neg_inf))
            m = jnp.max(mv)
            plsc.store_scatter(red_v, [_splat_i(ns)], _splat_f(m), mask=lane0)
            return carry

        lax.fori_loop(lo, hi + 1, _max_ns, 0)

        _reduce_chunks(jnp.maximum)

        def _e_group(g, carry):
            segg = segf_v[pl.ds(g * 16, 16)]
            sx = s_v[pl.ds(g * 16, 16)]
            mx = plsc.load_gather(red_v, [segg])
            e_v[pl.ds(g * 16, 16)] = jnp.exp(sx - mx)
            return carry

        lax.fori_loop(0, ngrp, _e_group, 0)

        for cc in range(_NS // 16):
            red_v[pl.ds(cc * 16, 16)] = jnp.zeros((16,), jnp.float32)

        def _sum_ns(ns, carry):
            sv = jnp.zeros((16,), jnp.float32)
            for g in range(ngrp):
                segg = segf_v[pl.ds(g * 16, 16)]
                ex = e_v[pl.ds(g * 16, 16)]
                sv = sv + jnp.where(segg == ns, ex, 0.0)
            t = jnp.sum(sv)
            plsc.store_scatter(red_v, [_splat_i(ns)], _splat_f(t), mask=lane0)
            return carry

        lax.fori_loop(lo, hi + 1, _sum_ns, 0)

        _reduce_chunks(jnp.add)

        def _a_group(g, carry):
            segg = segf_v[pl.ds(g * 16, 16)]
            ex = e_v[pl.ds(g * 16, 16)]
            es = plsc.load_gather(red_v, [segg])
            a_v[pl.ds(g * 16, 16)] = ex / es
            return carry

        lax.fori_loop(0, ngrp, _a_group, 0)

        pltpu.sync_copy(a_v, a_hbm.at[b, pl.ds(tok0, ctok)])

    return _sc_body, spc, nch, ctok, nrows


def _sc_weights(q, k, seg):
    nb = q.shape[0]
    body, spc, nch, ctok, nrows = _make_sc_body(nb)
    mesh = plsc.VectorSubcoreMesh(core_axis_name="c", subcore_axis_name="s")
    f = pl.kernel(
        body,
        out_type=jax.ShapeDtypeStruct((nb, _S), jnp.float32),
        mesh=mesh,
        compiler_params=pltpu.CompilerParams(needs_layout_passes=False),
        scratch_types=[
            pltpu.VMEM((ctok,), jnp.int32),
            pltpu.VMEM((nrows, 128), jnp.int32),
            pltpu.VMEM((ctok, _DQ), jnp.float32),
            pltpu.VMEM((_DQ, ctok), jnp.float32),
            pltpu.VMEM((_NS, _DQ), jnp.float32),
            pltpu.VMEM((ctok,), jnp.float32),
            pltpu.VMEM((ctok,), jnp.float32),
            pltpu.VMEM((_NS,), jnp.float32),
            pltpu.VMEM((nch, _NS), jnp.float32),
            pltpu.VMEM((ctok,), jnp.float32),
            pltpu.VMEM_SHARED((spc * _NS, _DQ), jnp.float32),
            pltpu.VMEM_SHARED((spc, nch, _NS), jnp.float32),
            pltpu.SemaphoreType.DMA,
            pltpu.SemaphoreType.DMA,
            pltpu.SemaphoreType.DMA,
        ],
    )
    return f(q, k, seg)


def _pool_body(seg_ref, a_ref, x_ref, out_ref):
    xb = x_ref[0]
    seg_row = seg_ref[0]
    a_row = a_ref[0]
    ids_ns_s = lax.broadcasted_iota(jnp.int32, (_NS, _S), 0)
    onehot = (ids_ns_s == seg_row).astype(jnp.float32)
    out_ref[0] = jnp.dot(onehot * a_row, xb, preferred_element_type=jnp.float32)


def _pool(x, seg3, a):
    nb = x.shape[0]
    return pl.pallas_call(
        _pool_body,
        grid=(nb,),
        in_specs=[
            pl.BlockSpec((1, 1, _S), lambda b: (b, 0, 0)),
            pl.BlockSpec((1, 1, _S), lambda b: (b, 0, 0)),
            pl.BlockSpec((1, _S, _D), lambda b: (b, 0, 0)),
        ],
        out_specs=pl.BlockSpec((1, _NS, _D), lambda b: (b, 0, 0)),
        out_shape=jax.ShapeDtypeStruct((nb, _NS, _D), jnp.float32),
        compiler_params=pltpu.CompilerParams(
            dimension_semantics=("arbitrary",),
        ),
    )(seg3, a.reshape(nb, 1, _S), x)


def kernel(x, sentence_index, Wq, bq, Wk, bk):
    seg = sentence_index.astype(jnp.int32)
    q, k = _projections(x, Wq, bq, Wk, bk)
    a = _sc_weights(q, k, seg)
    out = _pool(x, seg.reshape(_B, 1, _S), a)
    return out[:, 1:]


# speedup vs baseline: 1.2604x; 1.0015x over previous
import jax
import jax.numpy as jnp
from jax import lax
from jax.experimental import pallas as pl
from jax.experimental.pallas import tpu as pltpu
from jax.experimental.pallas import tpu_sc as plsc

_B, _S, _D = 8, 2048, 768
_NS = 64
_DQ = _D // 8
_CHUNK = 512
_NCH = _S // _CHUNK
_NGRP = _CHUNK // 16


def _proj_body(x_ref, wq_ref, bq_ref, wk_ref, bk_ref, qt_ref, k_ref):
    xb = x_ref[0]
    qt = lax.dot_general(wq_ref[:], xb, (((0,), (1,)), ((), ())),
                         preferred_element_type=jnp.float32)
    qt_ref[0] = qt + bq_ref[:]
    k_ref[0] = jnp.dot(xb, wk_ref[:], preferred_element_type=jnp.float32) + bk_ref[:]


def _projections(x, Wq, bq, Wk, bk):
    nb = x.shape[0]
    return pl.pallas_call(
        _proj_body,
        grid=(nb,),
        in_specs=[
            pl.BlockSpec((1, _S, _D), lambda b: (b, 0, 0)),
            pl.BlockSpec((_D, _DQ), lambda b: (0, 0)),
            pl.BlockSpec((_DQ, 1), lambda b: (0, 0)),
            pl.BlockSpec((_D, _DQ), lambda b: (0, 0)),
            pl.BlockSpec((1, _DQ), lambda b: (0, 0)),
        ],
        out_specs=[
            pl.BlockSpec((1, _DQ, _S), lambda b: (b, 0, 0)),
            pl.BlockSpec((1, _S, _DQ), lambda b: (b, 0, 0)),
        ],
        out_shape=[
            jax.ShapeDtypeStruct((nb, _DQ, _S), jnp.float32),
            jax.ShapeDtypeStruct((nb, _S, _DQ), jnp.float32),
        ],
        compiler_params=pltpu.CompilerParams(
            dimension_semantics=("arbitrary",),
        ),
    )(x, Wq, bq.reshape(_DQ, 1), Wk, bk.reshape(1, _DQ))


def _iota16():
    return lax.iota(jnp.int32, 16)


def _splat_f(v):
    return jnp.zeros((16,), jnp.float32) + v


def _splat_i(v):
    return jnp.zeros((16,), jnp.int32) + v


def _make_sc_body(nb):
    spc = nb // 2
    nch = 16 // spc
    ctok = _S // nch
    ngrp = ctok // 16
    nrows = ctok // 128

    def _sc_body(q_hbm, k_hbm, seg_hbm, a_hbm,
                 segf_v, seg2_v, kq_v, qt_v, ksum_v, s_v, e_v, red_v, part_v,
                 a_v, ksum_sh, stage_sh, sem0, sem1, sem2):
        cid = lax.axis_index("c")
        sid = lax.axis_index("s")
        slab = sid // nch
        chunk = sid % nch
        b = cid * spc + slab
        tok0 = chunk * ctok

        seg_cp = pltpu.async_copy(seg_hbm.at[b, pl.ds(tok0, ctok)], segf_v, sem0)
        k_cp = pltpu.async_copy(k_hbm.at[b, pl.ds(tok0, ctok), :], kq_v, sem1)
        q_cp = pltpu.async_copy(q_hbm.at[b, :, pl.ds(tok0, ctok)], qt_v, sem2)

        seg_cp.wait()
        for r in range(nrows):
            for cc in range(8):
                v = segf_v[pl.ds(r * 128 + cc * 16, 16)] + slab * _NS
                seg2_v[r, cc * 16:(cc + 1) * 16] = v

        @pl.when(chunk == 0)
        def _():
            for rr in range(_NS):
                for cc in range(_DQ // 16):
                    ksum_v[rr, cc * 16:(cc + 1) * 16] = jnp.zeros((16,), jnp.float32)
            pltpu.sync_copy(ksum_v, ksum_sh.at[pl.ds(slab * _NS, _NS), :])
        plsc.subcore_barrier()

        k_cp.wait()
        adds = [pltpu.async_copy(kq_v.at[pl.ds(r * 128, 128), :],
                                 ksum_sh.at[seg2_v.at[r]], sem1, add=True)
                for r in range(nrows)]
        for cp in adds:
            cp.wait()
        plsc.subcore_barrier()

        pltpu.sync_copy(ksum_sh.at[pl.ds(slab * _NS, _NS), :], ksum_v)

        q_cp.wait()

        def _s_group(g, carry):
            segg = segf_v[pl.ds(g * 16, 16)]
            acc = jnp.zeros((16,), jnp.float32)
            for j in range(_DQ):
                qv = qt_v[j, pl.ds(g * 16, 16)]
                kg = plsc.load_gather(ksum_v, [segg, _splat_i(j)])
                acc = acc + qv * kg
            s_v[pl.ds(g * 16, 16)] = acc
            return carry

        lax.fori_loop(0, ngrp, _s_group, 0)

        lo = jnp.min(segf_v[pl.ds(0, 16)])
        hi = jnp.max(segf_v[pl.ds((ngrp - 1) * 16, 16)])

        lane0 = _iota16() == 0
        neg_inf = jnp.float32(-jnp.inf)

        def _reduce_chunks(op):
            pltpu.sync_copy(red_v, stage_sh.at[slab, chunk])
            plsc.subcore_barrier()
            pltpu.sync_copy(stage_sh.at[slab], part_v)
            plsc.subcore_barrier()
            for cc in range(_NS // 16):
                v = part_v[0, cc * 16:(cc + 1) * 16]
                for r in range(1, nch):
                    v = op(v, part_v[r, cc * 16:(cc + 1) * 16])
                red_v[pl.ds(cc * 16, 16)] = v

        for cc in range(_NS // 16):
            red_v[pl.ds(cc * 16, 16)] = _splat_f(neg_inf)

        def _max_ns(ns, carry):
            mv = _splat_f(neg_inf)
            for g in range(ngrp):
                segg = segf_v[pl.ds(g * 16, 16)]
                sx = s_v[pl.ds(g * 16, 16)]
                mv = jnp.maximum(mv, jnp.where(segg == ns, sx, neg_inf))
            m = jnp.max(mv)
            plsc.store_scatter(red_v, [_splat_i(ns)], _splat_f(m), mask=lane0)
            return carry

        lax.fori_loop(lo, hi + 1, _max_ns, 0)

        _reduce_chunks(jnp.maximum)

        def _e_group(g, carry):
            segg = segf_v[pl.ds(g * 16, 16)]
            sx = s_v[pl.ds(g * 16, 16)]
            mx = plsc.load_gather(red_v, [segg])
            e_v[pl.ds(g * 16, 16)] = jnp.exp(sx - mx)
            return carry

        lax.fori_loop(0, ngrp, _e_group, 0)

        for cc in range(_NS // 16):
            red_v[pl.ds(cc * 16, 16)] = jnp.zeros((16,), jnp.float32)

        def _sum_ns(ns, carry):
            sv = jnp.zeros((16,), jnp.float32)
            for g in range(ngrp):
                segg = segf_v[pl.ds(g * 16, 16)]
                ex = e_v[pl.ds(g * 16, 16)]
                sv = sv + jnp.where(segg == ns, ex, 0.0)
            t = jnp.sum(sv)
            plsc.store_scatter(red_v, [_splat_i(ns)], _splat_f(t), mask=lane0)
            return carry

        lax.fori_loop(lo, hi + 1, _sum_ns, 0)

        _reduce_chunks(jnp.add)

        def _a_group(g, carry):
            segg = segf_v[pl.ds(g * 16, 16)]
            ex = e_v[pl.ds(g * 16, 16)]
            es = plsc.load_gather(red_v, [segg])
            a_v[pl.ds(g * 16, 16)] = ex / es
            return carry

        lax.fori_loop(0, ngrp, _a_group, 0)

        pltpu.sync_copy(a_v, a_hbm.at[b, pl.ds(tok0, ctok)])

    return _sc_body, spc, nch, ctok, nrows


def _sc_weights(q, k, seg):
    nb = q.shape[0]
    body, spc, nch, ctok, nrows = _make_sc_body(nb)
    mesh = plsc.VectorSubcoreMesh(core_axis_name="c", subcore_axis_name="s")
    f = pl.kernel(
        body,
        out_type=jax.ShapeDtypeStruct((nb, _S), jnp.float32),
        mesh=mesh,
        compiler_params=pltpu.CompilerParams(needs_layout_passes=False),
        scratch_types=[
            pltpu.VMEM((ctok,), jnp.int32),
            pltpu.VMEM((nrows, 128), jnp.int32),
            pltpu.VMEM((ctok, _DQ), jnp.float32),
            pltpu.VMEM((_DQ, ctok), jnp.float32),
            pltpu.VMEM((_NS, _DQ), jnp.float32),
            pltpu.VMEM((ctok,), jnp.float32),
            pltpu.VMEM((ctok,), jnp.float32),
            pltpu.VMEM((_NS,), jnp.float32),
            pltpu.VMEM((nch, _NS), jnp.float32),
            pltpu.VMEM((ctok,), jnp.float32),
            pltpu.VMEM_SHARED((spc * _NS, _DQ), jnp.float32),
            pltpu.VMEM_SHARED((spc, nch, _NS), jnp.float32),
            pltpu.SemaphoreType.DMA,
            pltpu.SemaphoreType.DMA,
            pltpu.SemaphoreType.DMA,
        ],
    )
    return f(q, k, seg)


def _pool_body(seg_ref, a_ref, x_ref, out_ref):
    xb = x_ref[0]
    seg_row = seg_ref[0]
    a_row = a_ref[0]
    ids_ns_s = lax.broadcasted_iota(jnp.int32, (_NS, _S), 0)
    onehot = (ids_ns_s == seg_row).astype(jnp.float32)
    out_ref[0] = jnp.dot(onehot * a_row, xb, preferred_element_type=jnp.float32)


def _pool(x, seg3, a):
    nb = x.shape[0]
    return pl.pallas_call(
        _pool_body,
        grid=(nb,),
        in_specs=[
            pl.BlockSpec((1, 1, _S), lambda b: (b, 0, 0)),
            pl.BlockSpec((1, 1, _S), lambda b: (b, 0, 0)),
            pl.BlockSpec((1, _S, _D), lambda b: (b, 0, 0)),
        ],
        out_specs=pl.BlockSpec((1, _NS, _D), lambda b: (b, 0, 0)),
        out_shape=jax.ShapeDtypeStruct((nb, _NS, _D), jnp.float32),
        compiler_params=pltpu.CompilerParams(
            dimension_semantics=("arbitrary",),
        ),
    )(seg3, a.reshape(nb, 1, _S), x)


def kernel(x, sentence_index, Wq, bq, Wk, bk):
    seg = sentence_index.astype(jnp.int32)
    q, k = _projections(x, Wq, bq, Wk, bk)
    a = _sc_weights(q, k, seg)
    out = _pool(x, seg.reshape(_B, 1, _S), a)
    return out[:, 1:]
